# baseline (speedup 1.0000x reference)
import functools

import jax
import jax.numpy as jnp
from jax import lax
from jax.experimental import pallas as pl
from jax.experimental.pallas import tpu as pltpu
from jax.experimental.pallas import tpu_sc as plsc

B = 4096
L = 200
D = 64
SCALE = float(D) ** 0.5
EPS = 1.1754943508222875e-38
N_ELEM = L * D

_NC = 2
_NS = 16
NW = _NC * _NS
BPW = B // NW

_mesh = plsc.VectorSubcoreMesh(core_axis_name="c", subcore_axis_name="s")


@functools.partial(
    pl.kernel,
    mesh=_mesh,
    compiler_params=pltpu.CompilerParams(use_tc_tiling_on_sc=False),
    out_type=[
        jax.ShapeDtypeStruct((L * B, D), jnp.float32),
        jax.ShapeDtypeStruct((B, D), jnp.float32),
        jax.ShapeDtypeStruct((B, D), jnp.float32),
    ],
    scratch_types=[
        pltpu.VMEM((L, BPW), jnp.int32),
        pltpu.VMEM((BPW, D), jnp.float32),
        pltpu.VMEM((BPW, D), jnp.float32),
        pltpu.VMEM((BPW, D), jnp.float32),
        pltpu.SemaphoreType.DMA,
    ],
)
def _sc_gather_stats(idx_hbm, emb_hbm, raw_hbm, s_hbm, q_hbm,
                     idx_v, rows_v, acc_s, acc_q, sem):
    wid = lax.axis_index("s") * _NC + lax.axis_index("c")
    b0 = wid * BPW

    pltpu.sync_copy(idx_hbm.at[wid], idx_v)

    zeros = jnp.zeros((16,), jnp.float32)

    def zero_body(r, _):
        for c in range(D // 16):
            acc_s[r, pl.ds(c * 16, 16)] = zeros
            acc_q[r, pl.ds(c * 16, 16)] = zeros
        return 0
    lax.fori_loop(0, BPW, zero_body, 0)

    def l_body(l, _):
        pltpu.async_copy(emb_hbm.at[idx_v.at[l]], rows_v, sem).wait()

        def r_body(r, _):
            for c in range(D // 16):
                x = rows_v[r, pl.ds(c * 16, 16)]
                acc_s[r, pl.ds(c * 16, 16)] += x
                acc_q[r, pl.ds(c * 16, 16)] += x * x
            return 0
        lax.fori_loop(0, BPW, r_body, 0)

        pltpu.sync_copy(rows_v, raw_hbm.at[pl.ds(l * B + b0, BPW)])
        return 0
    lax.fori_loop(0, L, l_body, 0)

    pltpu.sync_copy(acc_s, s_hbm.at[pl.ds(b0, BPW)])
    pltpu.sync_copy(acc_q, q_hbm.at[pl.ds(b0, BPW)])


def _finalize_body(s_ref, q_ref, a_ref, c_ref):
    s = s_ref[:, :]
    q = q_ref[:, :]
    sum_b = jnp.sum(s, axis=1, keepdims=True)
    sumsq_b = jnp.sum(q, axis=1, keepdims=True)
    n = jnp.float32(N_ELEM)
    mean = sum_b / n
    var = (sumsq_b - sum_b * sum_b / n) / (n - 1.0)
    std = jnp.sqrt(var)
    inv = SCALE / (SCALE * std + EPS)
    a_ref[:, :] = jnp.broadcast_to(inv, (B, D))
    c_ref[:, :] = jnp.broadcast_to(-mean * inv, (B, D))


_finalize = pl.pallas_call(
    _finalize_body,
    out_shape=[
        jax.ShapeDtypeStruct((B, D), jnp.float32),
        jax.ShapeDtypeStruct((B, D), jnp.float32),
    ],
)


def _norm_body(x_ref, a_ref, c_ref, o_ref):
    o_ref[...] = x_ref[...] * a_ref[...][None] + c_ref[...][None]


_COLS = B * D // 128
_LBLK = 4

_norm = pl.pallas_call(
    _norm_body,
    grid=(L // _LBLK,),
    in_specs=[
        pl.BlockSpec((_LBLK, _COLS, 128), lambda i: (i, 0, 0)),
        pl.BlockSpec((_COLS, 128), lambda i: (0, 0)),
        pl.BlockSpec((_COLS, 128), lambda i: (0, 0)),
    ],
    out_specs=pl.BlockSpec((_LBLK, _COLS, 128), lambda i: (i, 0, 0)),
    out_shape=jax.ShapeDtypeStruct((L, _COLS, 128), jnp.float32),
)


def kernel(inp, emb):
    idx_w = inp.reshape(NW, BPW, L).transpose(0, 2, 1)
    raw, s, q = _sc_gather_stats(idx_w, emb)
    a, c = _finalize(s, q)
    out = _norm(raw.reshape(L, _COLS, 128),
                a.reshape(_COLS, 128), c.reshape(_COLS, 128))
    return out.reshape(L, B, D)


# speedup vs baseline: 1.2065x; 1.2065x over previous
import functools

import jax
import jax.numpy as jnp
from jax import lax
from jax.experimental import pallas as pl
from jax.experimental.pallas import tpu as pltpu
from jax.experimental.pallas import tpu_sc as plsc

B = 4096
L = 200
D = 64
SCALE = float(D) ** 0.5
EPS = 1.1754943508222875e-38
N_ELEM = L * D

_NC = 2
_NS = 16
NW = _NC * _NS
BPW = B // NW

NBUF = 4

_mesh = plsc.VectorSubcoreMesh(core_axis_name="c", subcore_axis_name="s")


@functools.partial(
    pl.kernel,
    mesh=_mesh,
    compiler_params=pltpu.CompilerParams(use_tc_tiling_on_sc=False),
    out_type=[
        jax.ShapeDtypeStruct((L * B, D), jnp.float32),
        jax.ShapeDtypeStruct((B, D), jnp.float32),
        jax.ShapeDtypeStruct((B, D), jnp.float32),
    ],
    scratch_types=[
        pltpu.VMEM((L, BPW), jnp.int32),
        pltpu.VMEM((NBUF, BPW, D), jnp.float32),
        pltpu.VMEM((BPW, D), jnp.float32),
        pltpu.VMEM((BPW, D), jnp.float32),
        pltpu.SemaphoreType.DMA((NBUF,)),
        pltpu.SemaphoreType.DMA((NBUF,)),
    ],
)
def _sc_gather_stats(idx_hbm, emb_hbm, raw_hbm, s_hbm, q_hbm,
                     idx_v, rows_v, acc_s, acc_q, gsem, wsem):
    wid = lax.axis_index("s") * _NC + lax.axis_index("c")
    b0 = wid * BPW

    pltpu.sync_copy(idx_hbm.at[wid], idx_v)

    zeros = jnp.zeros((16,), jnp.float32)

    def zero_body(r, _):
        for c in range(D // 16):
            acc_s[r, pl.ds(c * 16, 16)] = zeros
            acc_q[r, pl.ds(c * 16, 16)] = zeros
        return 0
    lax.fori_loop(0, BPW, zero_body, 0)

    def fire_gather(l, j):
        pltpu.async_copy(emb_hbm.at[idx_v.at[l]], rows_v.at[j], gsem.at[j])

    def fire_write(l, j):
        pltpu.async_copy(rows_v.at[j], raw_hbm.at[pl.ds(l * B + b0, BPW)],
                         wsem.at[j])

    def wait_gather(j):
        pltpu.make_async_copy(emb_hbm.at[idx_v.at[0]], rows_v.at[j],
                              gsem.at[j]).wait()

    def wait_write(j):
        pltpu.make_async_copy(rows_v.at[j], raw_hbm.at[pl.ds(b0, BPW)],
                              wsem.at[j]).wait()

    def accumulate(j):
        def r_body(r, _):
            for c in range(D // 16):
                x = rows_v[j, r, pl.ds(c * 16, 16)]
                acc_s[r, pl.ds(c * 16, 16)] += x
                acc_q[r, pl.ds(c * 16, 16)] += x * x
            return 0
        lax.fori_loop(0, BPW, r_body, 0)

    for j in range(NBUF - 1):
        fire_gather(j, j)

    def step(l, j, jprev, first):
        wait_gather(j)
        fire_write(l, j)
        accumulate(j)
        if not first:
            wait_write(jprev)
        fire_gather(l + NBUF - 1, jprev)

    step(0, 0, NBUF - 1, True)

    for l in range(1, NBUF):
        step(l, l % NBUF, (l - 1) % NBUF, False)

    def chunk(c, _):
        base = c * NBUF
        for j in range(NBUF):
            step(base + j, j, (j - 1) % NBUF, False)
        return 0
    lax.fori_loop(1, (L // NBUF) - 1, chunk, 0)

    l = 196
    wait_gather(l % NBUF)
    fire_write(l, l % NBUF)
    accumulate(l % NBUF)
    wait_write((l - 1) % NBUF)
    fire_gather(199, (l - 1) % NBUF)
    for l in (197, 198, 199):
        wait_gather(l % NBUF)
        fire_write(l, l % NBUF)
        accumulate(l % NBUF)

    for j in range(NBUF):
        wait_write(j)

    pltpu.sync_copy(acc_s, s_hbm.at[pl.ds(b0, BPW)])
    pltpu.sync_copy(acc_q, q_hbm.at[pl.ds(b0, BPW)])


def _finalize_body(s_ref, q_ref, a_ref, c_ref):
    s = s_ref[:, :]
    q = q_ref[:, :]
    sum_b = jnp.sum(s, axis=1, keepdims=True)
    sumsq_b = jnp.sum(q, axis=1, keepdims=True)
    n = jnp.float32(N_ELEM)
    mean = sum_b / n
    var = (sumsq_b - sum_b * sum_b / n) / (n - 1.0)
    std = jnp.sqrt(var)
    inv = SCALE / (SCALE * std + EPS)
    a_ref[:, :] = jnp.broadcast_to(inv, (B, D))
    c_ref[:, :] = jnp.broadcast_to(-mean * inv, (B, D))


_finalize = pl.pallas_call(
    _finalize_body,
    out_shape=[
        jax.ShapeDtypeStruct((B, D), jnp.float32),
        jax.ShapeDtypeStruct((B, D), jnp.float32),
    ],
)


def _norm_body(x_ref, a_ref, c_ref, o_ref):
    o_ref[...] = x_ref[...] * a_ref[...][None] + c_ref[...][None]


_COLS = B * D // 128
_LBLK = 4

_norm = pl.pallas_call(
    _norm_body,
    grid=(L // _LBLK,),
    in_specs=[
        pl.BlockSpec((_LBLK, _COLS, 128), lambda i: (i, 0, 0)),
        pl.BlockSpec((_COLS, 128), lambda i: (0, 0)),
        pl.BlockSpec((_COLS, 128), lambda i: (0, 0)),
    ],
    out_specs=pl.BlockSpec((_LBLK, _COLS, 128), lambda i: (i, 0, 0)),
    out_shape=jax.ShapeDtypeStruct((L, _COLS, 128), jnp.float32),
)


def kernel(inp, emb):
    idx_w = inp.reshape(NW, BPW, L).transpose(0, 2, 1)
    raw, s, q = _sc_gather_stats(idx_w, emb)
    a, c = _finalize(s, q)
    out = _norm(raw.reshape(L, _COLS, 128),
                a.reshape(_COLS, 128), c.reshape(_COLS, 128))
    return out.reshape(L, B, D)
